# Initial kernel scaffold; baseline (speedup 1.0000x reference)
import jax
import jax.numpy as jnp
from jax.experimental import pallas as pl


def kernel(logits):
    raise NotImplementedError("write your pallas kernel here")


# speedup vs baseline: 26.0363x; 26.0363x over previous
import math

import jax
import jax.numpy as jnp
from jax.experimental import pallas as pl

_I32_MIN = -(2**31)


def _topk_mask_body(x_ref, o_ref, *, k):
    x = x_ref[...]
    b = jax.lax.bitcast_convert_type(x, jnp.int32)
    key = jnp.where(b < 0, _I32_MIN - b, b)

    rows = x.shape[0]
    t_u = jnp.zeros((rows, 1), jnp.int32)

    def bit_step(i, t_u):
        bit = 31 - i
        cand_u = t_u | jax.lax.shift_left(jnp.ones((), jnp.int32), bit)
        cand_s = cand_u ^ _I32_MIN
        cnt = jnp.sum((key >= cand_s).astype(jnp.int32), axis=1, keepdims=True)
        return jnp.where(cnt >= k, cand_u, t_u)

    t_u = jax.lax.fori_loop(0, 32, bit_step, t_u)
    t_s = t_u ^ _I32_MIN

    gt = key > t_s
    eq = key == t_s
    c_gt = jnp.sum(gt.astype(jnp.int32), axis=1, keepdims=True)
    e = k - c_gt

    idx = jax.lax.broadcasted_iota(jnp.int32, x.shape, 1)
    eq_i = eq.astype(jnp.int32)
    t_i = jnp.zeros((rows, 1), jnp.int32)

    def idx_step(i, t_i):
        bit = 12 - i
        cand = t_i + jax.lax.shift_left(jnp.ones((), jnp.int32), bit)
        cnt = jnp.sum(jnp.where(idx < cand, eq_i, 0), axis=1, keepdims=True)
        return jnp.where(cnt < e, cand, t_i)

    t_i = jax.lax.fori_loop(0, 13, idx_step, t_i)
    keep = gt | (eq & (idx <= t_i))
    o_ref[...] = jnp.where(keep, x, -jnp.inf)


def kernel(logits):
    B, S, V = logits.shape
    k = math.ceil((1.0 - 0.9) * V)
    n = B * S
    x = logits.reshape(n, V)
    rows_per_block = 64
    grid = (n // rows_per_block,)
    out = pl.pallas_call(
        lambda x_ref, o_ref: _topk_mask_body(x_ref, o_ref, k=k),
        grid=grid,
        in_specs=[pl.BlockSpec((rows_per_block, V), lambda i: (i, 0))],
        out_specs=pl.BlockSpec((rows_per_block, V), lambda i: (i, 0)),
        out_shape=jax.ShapeDtypeStruct((n, V), jnp.float32),
    )(x)
    return out.reshape(B, S, V)
